# Initial kernel scaffold; baseline (speedup 1.0000x reference)
import jax
import jax.numpy as jnp
from jax.experimental import pallas as pl


def kernel(features, edge_index, norm_A, W0, b0, Wc, W_out, b_out):
    raise NotImplementedError("write your pallas kernel here")


# speedup vs baseline: 3.7923x; 3.7923x over previous
import functools
import math

import jax
import jax.numpy as jnp
from jax import lax
from jax.experimental import pallas as pl
from jax.experimental.pallas import tpu as pltpu
from jax.experimental.pallas import tpu_sc as plsc

_ALPHA = 0.1
_LAMDA = 0.5

_NC = 2
_NS = 16
_LANES = 16
_NW = _NC * _NS
_C = 80


def _tc_in_body(f_ref, w_ref, b_ref, o_ref):
    acc = jnp.dot(f_ref[...], w_ref[...], preferred_element_type=jnp.float32,
                  precision=lax.Precision.HIGHEST)
    o_ref[...] = jnp.maximum(acc + b_ref[...], 0.0)


def _tc_layer_body(parts_ref, h0_ref, w_ref, o_ref):
    s = parts_ref[0] + parts_ref[1]
    acc = jnp.dot(s, w_ref[0], preferred_element_type=jnp.float32,
                  precision=lax.Precision.HIGHEST)
    acc = acc + jnp.dot(h0_ref[...], w_ref[1], preferred_element_type=jnp.float32,
                  precision=lax.Precision.HIGHEST)
    o_ref[...] = jnp.maximum(acc, 0.0)


def _tc_out_body(x_ref, w_ref, b_ref, o_ref):
    acc = jnp.dot(x_ref[...], w_ref[...], preferred_element_type=jnp.float32,
                  precision=lax.Precision.HIGHEST)
    o_ref[...] = acc + b_ref[...]


def _row_block(n):
    for rb in (2000, 1000, 500, 200, 100, 50, 8):
        if n % rb == 0:
            return rb
    return n


def _tc_in(features, w0, b0, interpret=False):
    n, din = features.shape
    dh = w0.shape[1]
    rb = _row_block(n)
    return pl.pallas_call(
        _tc_in_body,
        grid=(n // rb,),
        in_specs=[
            pl.BlockSpec((rb, din), lambda i: (i, 0)),
            pl.BlockSpec((din, dh), lambda i: (0, 0)),
            pl.BlockSpec((1, dh), lambda i: (0, 0)),
        ],
        out_specs=pl.BlockSpec((rb, dh), lambda i: (i, 0)),
        out_shape=jax.ShapeDtypeStruct((n, dh), jnp.float32),
        interpret=interpret,
    )(features, w0, b0.reshape(1, dh))


def _tc_layer(parts, h0, wstack, interpret=False):
    n, dh = h0.shape
    rb = _row_block(n)
    return pl.pallas_call(
        _tc_layer_body,
        grid=(n // rb,),
        in_specs=[
            pl.BlockSpec((2, rb, dh), lambda i: (0, i, 0)),
            pl.BlockSpec((rb, dh), lambda i: (i, 0)),
            pl.BlockSpec((2, dh, dh), lambda i: (0, 0, 0)),
        ],
        out_specs=pl.BlockSpec((rb, dh), lambda i: (i, 0)),
        out_shape=jax.ShapeDtypeStruct((n, dh), jnp.float32),
        interpret=interpret,
    )(parts, h0, wstack)


def _tc_out(x, w_out, b_out, interpret=False):
    n, dh = x.shape
    dc = w_out.shape[1]
    rb = _row_block(n)
    return pl.pallas_call(
        _tc_out_body,
        grid=(n // rb,),
        in_specs=[
            pl.BlockSpec((rb, dh), lambda i: (i, 0)),
            pl.BlockSpec((dh, dc), lambda i: (0, 0)),
            pl.BlockSpec((1, dc), lambda i: (0, 0)),
        ],
        out_specs=pl.BlockSpec((rb, dc), lambda i: (i, 0)),
        out_shape=jax.ShapeDtypeStruct((n, dc), jnp.float32),
        interpret=interpret,
    )(x, w_out, b_out.reshape(1, dc))


def _make_spmm(n_nodes, d, n_edges, interpret=False):
    assert n_edges % _NW == 0
    ept = n_edges // _NW
    assert ept % _C == 0
    nchunks = ept // _C
    rpt = (n_nodes // _NS) // 8 * 8
    tail = n_nodes - rpt * _NS
    assert tail % 8 == 0 and tail <= rpt
    groups = _C // _LANES
    kk = d // _LANES

    mesh = plsc.VectorSubcoreMesh(core_axis_name="c", subcore_axis_name="s",
                                  num_cores=_NC, num_subcores=_NS)

    def body(x_hbm, src_hbm, dst_hbm, norm_hbm, zeros_hbm, out_hbm,
             src_v, dst_v, norm_v, rows_v, hi_sh, sem):
        cid = lax.axis_index("c")
        sid = lax.axis_index("s")
        wid = cid * _NS + sid
        pltpu.sync_copy(zeros_hbm, hi_sh.at[pl.ds(sid * rpt, rpt)])
        if tail:
            @pl.when(sid == _NS - 1)
            def _zero_tail():
                pltpu.sync_copy(zeros_hbm.at[pl.ds(0, tail)],
                                hi_sh.at[pl.ds(_NS * rpt, tail)])
        plsc.subcore_barrier()
        ebase = wid * ept

        def chunk(c, carry):
            base = ebase + c * _C
            pltpu.sync_copy(src_hbm.at[pl.ds(base, _C)], src_v)
            pltpu.sync_copy(dst_hbm.at[pl.ds(base, _C)], dst_v)
            pltpu.sync_copy(norm_hbm.at[pl.ds(base, _C)], norm_v)
            pltpu.async_copy(x_hbm.at[src_v], rows_v, sem).wait()

            def group(g, carry2):
                nv = norm_v[pl.ds(g * _LANES, _LANES)]
                dnums = lax.GatherDimensionNumbers(
                    offset_dims=(), collapsed_slice_dims=(0,),
                    start_index_map=(0,))
                for e in range(_LANES):
                    s16 = lax.gather(
                        nv, jnp.full((_LANES, 1), e, jnp.int32), dnums,
                        slice_sizes=(1,),
                        mode=lax.GatherScatterMode.PROMISE_IN_BOUNDS)
                    row = g * _LANES + e
                    for k in range(kk):
                        sl = pl.ds(k * _LANES, _LANES)
                        rows_v[row, sl] = rows_v[row, sl] * s16
                return carry2

            lax.fori_loop(0, groups, group, 0)
            pltpu.sync_copy(rows_v, hi_sh.at[dst_v], add=True)
            return carry

        lax.fori_loop(0, nchunks, chunk, 0)
        plsc.subcore_barrier()
        pltpu.sync_copy(hi_sh.at[pl.ds(sid * rpt, rpt)],
                        out_hbm.at[cid, pl.ds(sid * rpt, rpt)])
        if tail:
            @pl.when(sid == _NS - 1)
            def _write_tail():
                pltpu.sync_copy(hi_sh.at[pl.ds(_NS * rpt, tail)],
                                out_hbm.at[cid, pl.ds(_NS * rpt, tail)])

    return pl.kernel(
        body,
        out_type=jax.ShapeDtypeStruct((_NC, n_nodes, d), jnp.float32),
        mesh=mesh,
        scratch_types=[
            pltpu.VMEM((_C,), jnp.int32),
            pltpu.VMEM((_C,), jnp.int32),
            pltpu.VMEM((_C,), jnp.float32),
            pltpu.VMEM((_C, d), jnp.float32),
            pltpu.VMEM_SHARED((n_nodes, d), jnp.float32),
            pltpu.SemaphoreType.DMA,
        ],
        interpret=interpret,
    )


def _gcnii(features, edge_index, norm_A, W0, b0, Wc, W_out, b_out,
           interpret=False):
    n, _ = features.shape
    dh = W0.shape[1]
    e = norm_A.shape[0]
    nl = Wc.shape[0]
    src = edge_index[0]
    dst = edge_index[1]
    zeros_slab = jnp.zeros(((n // _NS) // 8 * 8, dh), jnp.float32)

    x = _tc_in(features, W0, b0, interpret=interpret)
    h0 = x
    spmm = _make_spmm(n, dh, e, interpret=interpret)
    eye = jnp.eye(dh, dtype=jnp.float32)
    for i in range(nl):
        theta = math.log(_LAMDA / (i + 1) + 1.0)
        wp = theta * Wc[i] + (1.0 - theta) * eye
        wstack = jnp.stack([(1.0 - _ALPHA) * wp, _ALPHA * wp])
        parts = spmm(x, src, dst, norm_A, zeros_slab)
        x = _tc_layer(parts, h0, wstack, interpret=interpret)
    return _tc_out(x, W_out, b_out, interpret=interpret)


def kernel(features, edge_index, norm_A, W0, b0, Wc, W_out, b_out):
    return _gcnii(features, edge_index, norm_A, W0, b0, Wc, W_out, b_out)
